# baseline (speedup 1.0000x reference)
import functools
import jax
import jax.numpy as jnp
from jax import lax
from jax.experimental import pallas as pl
from jax.experimental.pallas import tpu as pltpu, tpu_sc as plsc

_CH = 8192


def kernel(sample):
    B, C, T = sample.shape
    NC, NS = 2, 16
    NW = NC * NS
    cols_per_w = T // NW
    n_sub = cols_per_w // _CH
    x = sample.reshape(B * C, T)

    mesh = plsc.VectorSubcoreMesh(core_axis_name="c", subcore_axis_name="s", num_cores=NC, num_subcores=NS)

    @functools.partial(
        pl.kernel,
        out_type=jax.ShapeDtypeStruct((B * 2, T), jnp.float32),
        mesh=mesh,
        scratch_types=[
            pltpu.VMEM((_CH,), jnp.float32),
            pltpu.VMEM((_CH,), jnp.float32),
            pltpu.VMEM((_CH,), jnp.float32),
            pltpu.VMEM((_CH,), jnp.float32),
        ],
    )
    def mix(x_hbm, out_hbm, buf_a, buf_b, buf_c, buf_o):
        wid = lax.axis_index("s") * NC + lax.axis_index("c")
        col0 = wid * cols_per_w

        def body(it, _):
            b = it // n_sub
            sub = it % n_sub
            off = pl.multiple_of(col0 + sub * _CH, _CH)
            pltpu.sync_copy(x_hbm.at[4 * b + 0, pl.ds(off, _CH)], buf_a)
            pltpu.sync_copy(x_hbm.at[4 * b + 1, pl.ds(off, _CH)], buf_b)
            pltpu.sync_copy(x_hbm.at[4 * b + 2, pl.ds(off, _CH)], buf_c)

            @plsc.parallel_loop(0, _CH, 16, unroll=8)
            def compute(i):
                buf_o[pl.ds(i, 16)] = (
                    buf_a[pl.ds(i, 16)] + buf_b[pl.ds(i, 16)] + buf_c[pl.ds(i, 16)]
                )

            pltpu.sync_copy(buf_o, out_hbm.at[2 * b + 0, pl.ds(off, _CH)])
            pltpu.sync_copy(x_hbm.at[4 * b + 3, pl.ds(off, _CH)], buf_a)
            pltpu.sync_copy(buf_a, out_hbm.at[2 * b + 1, pl.ds(off, _CH)])
            return 0

        lax.fori_loop(0, B * n_sub, body, 0)

    out = mix(x)
    return out.reshape(B, 2, T)
